# baseline (speedup 1.0000x reference)
import functools

import jax
import jax.numpy as jnp
from jax import lax
from jax.experimental import pallas as pl
from jax.experimental.pallas import tpu as pltpu
from jax.experimental.pallas import tpu_sc as plsc

F = 26
V = 100000
D = 32
B = 4096
DENSE_IN = 13
LR = 512
NL = 3
CROSS_IN = (F + 1) * D
S_DIM = F * D

_NC = 2
_NS = 16
_NW = _NC * _NS
_CHUNK = (B * F) // _NW
_IPS = 128
_KCH = _CHUNK // _IPS


def _sc_gather_body(idx_hbm, table_hbm, out_hbm, idx_v, rows_v, sem):
    wid = lax.axis_index("s") * _NC + lax.axis_index("c")
    base = wid * _CHUNK
    pltpu.sync_copy(idx_hbm.at[wid], idx_v)
    copies = [
        pltpu.async_copy(
            table_hbm.at[idx_v.at[j]],
            rows_v.at[pl.ds(j * _IPS, _IPS)],
            sem,
        )
        for j in range(_KCH)
    ]
    for c in copies:
        c.wait()
    pltpu.sync_copy(rows_v, out_hbm.at[pl.ds(base, _CHUNK)])


@functools.cache
def _sc_gather():
    return pl.kernel(
        _sc_gather_body,
        out_type=jax.ShapeDtypeStruct((B * F, D), jnp.float32),
        mesh=plsc.VectorSubcoreMesh(core_axis_name="c", subcore_axis_name="s"),
        scratch_types=[
            pltpu.VMEM((_KCH, _IPS), jnp.int32),
            pltpu.VMEM((_CHUNK, D), jnp.float32),
            pltpu.SemaphoreType.DMA,
        ],
        compiler_params=pltpu.CompilerParams(use_tc_tiling_on_sc=False),
    )


def _mm(a, w):
    return lax.dot_general(
        a.astype(jnp.bfloat16),
        w.astype(jnp.bfloat16),
        (((1,), (1,)), ((), ())),
        preferred_element_type=jnp.float32,
    )


def _dense_body(x_ref, s_ref, dw1, db1, dw2, db2, dw3, db3,
                vd, vs, wd, ws, bd, bs,
                o1d, o1s, ob1, ow2, ob2, ow3, ob3, out_ref):
    zero = jnp.float32(0.0)
    x = x_ref[...]
    h = jnp.maximum(_mm(x, dw1[...]) + db1[...], zero)
    h = jnp.maximum(_mm(h, dw2[...]) + db2[...], zero)
    d = jnp.maximum(_mm(h, dw3[...]) + db3[...], zero)
    s = s_ref[...]
    xld, xls = d, s
    for l in range(NL):
        xv = _mm(xld, vd[l]) + _mm(xls, vs[l])
        xld = d * (_mm(xv, wd[l]) + bd[l]) + xld
        xls = s * (_mm(xv, ws[l]) + bs[l]) + xls
    h = jnp.maximum(_mm(xld, o1d[...]) + _mm(xls, o1s[...]) + ob1[...], zero)
    h = jnp.maximum(_mm(h, ow2[...]) + ob2[...], zero)
    out_ref[...] = jnp.sum(h * ow3[...], axis=1, keepdims=True) + ob3[...]


_BB = 512
_GRID = B // _BB


def _full(shape):
    return pl.BlockSpec(shape, lambda i: (0,) * len(shape))


_dense_call = pl.pallas_call(
    _dense_body,
    grid=(_GRID,),
    in_specs=[
        pl.BlockSpec((_BB, DENSE_IN), lambda i: (i, 0)),
        pl.BlockSpec((_BB, S_DIM), lambda i: (i, 0)),
        _full((512, DENSE_IN)), _full((1, 512)),
        _full((256, 512)), _full((1, 256)),
        _full((D, 256)), _full((1, D)),
        _full((NL, LR, D)), _full((NL, LR, S_DIM)),
        _full((NL, D, LR)), _full((NL, S_DIM, LR)),
        _full((NL, 1, D)), _full((NL, 1, S_DIM)),
        _full((512, D)), _full((512, S_DIM)), _full((1, 512)),
        _full((256, 512)), _full((1, 256)),
        _full((1, 256)), _full((1, 1)),
    ],
    out_specs=pl.BlockSpec((_BB, 1), lambda i: (i, 0)),
    out_shape=jax.ShapeDtypeStruct((B, 1), jnp.float32),
)


def kernel(dense_features, sparse_indices, tables, dw1, db1, dw2, db2, dw3,
           db3, cnV, cnW, cnB, ow1, ob1, ow2, ob2, ow3, ob3):
    offs = (jnp.arange(F, dtype=jnp.int32) * V)[None, :]
    flat_idx = (sparse_indices + offs).reshape(_NW, _KCH, _IPS)
    table_flat = tables.reshape(F * V, D)
    sparse = _sc_gather()(flat_idx, table_flat)
    s = sparse.reshape(B, S_DIM)

    vd, vs = cnV[:, :, :D], cnV[:, :, D:]
    wd, ws = cnW[:, :D, :], cnW[:, D:, :]
    bd, bs = cnB[:, None, :D], cnB[:, None, D:]
    o1d, o1s = ow1[:, :D], ow1[:, D:]

    logits = _dense_call(
        dense_features, s,
        dw1, db1[None, :], dw2, db2[None, :], dw3, db3[None, :],
        vd, vs, wd, ws, bd, bs,
        o1d, o1s, ob1[None, :], ow2, ob2[None, :], ow3, ob3[None, :],
    )
    return logits


# speedup vs baseline: 4.6631x; 4.6631x over previous
import functools

import jax
import jax.numpy as jnp
from jax import lax
from jax.experimental import pallas as pl
from jax.experimental.pallas import tpu as pltpu
from jax.experimental.pallas import tpu_sc as plsc

F = 26
V = 100000
D = 32
B = 4096
DENSE_IN = 13
LR = 512
NL = 3
CROSS_IN = (F + 1) * D
S_DIM = F * D

_NC = 2
_NS = 16
_NW = _NC * _NS
_GROUPS = B // 16


def _sc_gather_body(idx_hbm, table_hbm, out_hbm, buf, idx_v, res_v):
    w = lax.axis_index("s") * _NC + lax.axis_index("c")

    def per_field(f, _):
        pltpu.sync_copy(table_hbm.at[f, w], buf)
        pltpu.sync_copy(idx_hbm.at[f], idx_v)

        def per_group(g, _):
            idx16 = idx_v[pl.ds(g * 16, 16)]
            res_v[pl.ds(g * 16, 16)] = plsc.load_gather(buf, [idx16])
            return 0

        lax.fori_loop(0, _GROUPS, per_group, 0)
        pltpu.sync_copy(res_v, out_hbm.at[f * D + w])
        return 0

    lax.fori_loop(0, F, per_field, 0)


@functools.cache
def _sc_gather():
    return pl.kernel(
        _sc_gather_body,
        out_type=jax.ShapeDtypeStruct((S_DIM, B), jnp.float32),
        mesh=plsc.VectorSubcoreMesh(core_axis_name="c", subcore_axis_name="s"),
        scratch_types=[
            pltpu.VMEM((V,), jnp.float32),
            pltpu.VMEM((B,), jnp.int32),
            pltpu.VMEM((B,), jnp.float32),
        ],
        compiler_params=pltpu.CompilerParams(needs_layout_passes=False),
    )


def _mmT(w, x):
    return lax.dot_general(
        w.astype(jnp.bfloat16),
        x.astype(jnp.bfloat16),
        (((1,), (0,)), ((), ())),
        preferred_element_type=jnp.float32,
    )


def _dense_body(x_ref, s_ref, dw1, db1, dw2, db2, dw3, db3,
                vd, vs, wd, ws, bd, bs,
                o1d, o1s, ob1, ow2, ob2, ow3t, ob3, out_ref):
    zero = jnp.float32(0.0)
    x = x_ref[...]
    h = jnp.maximum(_mmT(dw1[...], x) + db1[...], zero)
    h = jnp.maximum(_mmT(dw2[...], h) + db2[...], zero)
    d = jnp.maximum(_mmT(dw3[...], h) + db3[...], zero)
    s = s_ref[...]
    xld, xls = d, s
    for l in range(NL):
        xv = _mmT(vd[l], xld) + _mmT(vs[l], xls)
        xld = d * (_mmT(wd[l], xv) + bd[l]) + xld
        xls = s * (_mmT(ws[l], xv) + bs[l]) + xls
    h = jnp.maximum(_mmT(o1d[...], xld) + _mmT(o1s[...], xls) + ob1[...],
                    zero)
    h = jnp.maximum(_mmT(ow2[...], h) + ob2[...], zero)
    out_ref[...] = jnp.sum(h * ow3t[...], axis=0, keepdims=True) + ob3[...]


_BB = 512
_GRID = B // _BB


def _full(shape):
    return pl.BlockSpec(shape, lambda i: (0,) * len(shape))


_dense_call = pl.pallas_call(
    _dense_body,
    grid=(_GRID,),
    in_specs=[
        pl.BlockSpec((DENSE_IN, _BB), lambda i: (0, i)),
        pl.BlockSpec((S_DIM, _BB), lambda i: (0, i)),
        _full((512, DENSE_IN)), _full((512, 1)),
        _full((256, 512)), _full((256, 1)),
        _full((D, 256)), _full((D, 1)),
        _full((NL, LR, D)), _full((NL, LR, S_DIM)),
        _full((NL, D, LR)), _full((NL, S_DIM, LR)),
        _full((NL, D, 1)), _full((NL, S_DIM, 1)),
        _full((512, D)), _full((512, S_DIM)), _full((512, 1)),
        _full((256, 512)), _full((256, 1)),
        _full((256, 1)), _full((1, 1)),
    ],
    out_specs=pl.BlockSpec((1, _BB), lambda i: (0, i)),
    out_shape=jax.ShapeDtypeStruct((1, B), jnp.float32),
)


def kernel(dense_features, sparse_indices, tables, dw1, db1, dw2, db2, dw3,
           db3, cnV, cnW, cnB, ow1, ob1, ow2, ob2, ow3, ob3):
    idx_t = sparse_indices.T
    tables_t = tables.transpose(0, 2, 1)
    s_t = _sc_gather()(idx_t, tables_t)

    vd, vs = cnV[:, :, :D], cnV[:, :, D:]
    wd, ws = cnW[:, :D, :], cnW[:, D:, :]
    bd, bs = cnB[:, :D, None], cnB[:, D:, None]
    o1d, o1s = ow1[:, :D], ow1[:, D:]

    logits_t = _dense_call(
        dense_features.T, s_t,
        dw1, db1[:, None], dw2, db2[:, None], dw3, db3[:, None],
        vd, vs, wd, ws, bd, bs,
        o1d, o1s, ob1[:, None], ow2, ob2[:, None], ow3.T, ob3[:, None],
    )
    return logits_t.reshape(B, 1)


# speedup vs baseline: 5.3362x; 1.1444x over previous
import functools

import jax
import jax.numpy as jnp
from jax import lax
from jax.experimental import pallas as pl
from jax.experimental.pallas import tpu as pltpu
from jax.experimental.pallas import tpu_sc as plsc

F = 26
V = 100000
D = 32
B = 4096
DENSE_IN = 13
LR = 512
NL = 3
CROSS_IN = (F + 1) * D
S_DIM = F * D

_NC = 2
_NS = 16
_NW = _NC * _NS
_GROUPS = B // 16


_H0 = 49920
_H1 = V - _H0


def _sc_gather_body(idx_hbm, table_hbm, out_hbm,
                    bufa, bufb, idxa, idxb, resa, resb,
                    rs0, rs1, is0, is1, os0, os1):
    w = lax.axis_index("s") * _NC + lax.axis_index("c")
    bufs, rsem = (bufa, bufb), (rs0, rs1)
    idxv, isem = (idxa, idxb), (is0, is1)
    resv, osem = (resa, resb), (os0, os1)

    def start_row(k):
        f, h = divmod(k, 2)
        off, ln = (0, _H0) if h == 0 else (_H0, _H1)
        return pltpu.async_copy(
            table_hbm.at[f, w, pl.ds(off, ln)],
            bufs[k % 2].at[pl.ds(0, ln)], rsem[k % 2])

    def start_idx(f):
        return pltpu.async_copy(idx_hbm.at[f], idxv[f % 2], isem[f % 2])

    pend_row = {0: start_row(0)}
    pend_idx = {0: start_idx(0)}
    pend_out = {}
    for k in range(2 * F):
        f, h = divmod(k, 2)
        if k + 1 < 2 * F:
            pend_row[k + 1] = start_row(k + 1)
        if h == 0 and f + 1 < F:
            pend_idx[f + 1] = start_idx(f + 1)
        pend_row.pop(k).wait()
        if h == 0:
            pend_idx.pop(f).wait()
            if f >= 2:
                pend_out.pop(f - 2).wait()
        buf, iv, rv = bufs[k % 2], idxv[f % 2], resv[f % 2]

        if h == 0:
            def body0(g, _, iv=iv, rv=rv, buf=buf):
                idx16 = iv[pl.ds(g * 16, 16)]
                m = idx16 < _H0
                rv[pl.ds(g * 16, 16)] = plsc.load_gather(buf, [idx16], mask=m)
                return 0
            lax.fori_loop(0, _GROUPS, body0, 0)
        else:
            def body1(g, _, iv=iv, rv=rv, buf=buf):
                idx16 = iv[pl.ds(g * 16, 16)]
                m = idx16 >= _H0
                gv = plsc.load_gather(buf, [idx16 - _H0], mask=m)
                rv[pl.ds(g * 16, 16)] = jnp.where(m, gv, rv[pl.ds(g * 16, 16)])
                return 0
            lax.fori_loop(0, _GROUPS, body1, 0)
            pend_out[f] = pltpu.async_copy(rv, out_hbm.at[f * D + w],
                                           osem[f % 2])
    pend_out.pop(F - 2).wait()
    pend_out.pop(F - 1).wait()


@functools.cache
def _sc_gather():
    return pl.kernel(
        _sc_gather_body,
        out_type=jax.ShapeDtypeStruct((S_DIM, B), jnp.float32),
        mesh=plsc.VectorSubcoreMesh(core_axis_name="c", subcore_axis_name="s"),
        scratch_types=[
            pltpu.VMEM((_H1,), jnp.float32),
            pltpu.VMEM((_H1,), jnp.float32),
            pltpu.VMEM((B,), jnp.int32),
            pltpu.VMEM((B,), jnp.int32),
            pltpu.VMEM((B,), jnp.float32),
            pltpu.VMEM((B,), jnp.float32),
            pltpu.SemaphoreType.DMA,
            pltpu.SemaphoreType.DMA,
            pltpu.SemaphoreType.DMA,
            pltpu.SemaphoreType.DMA,
            pltpu.SemaphoreType.DMA,
            pltpu.SemaphoreType.DMA,
        ],
        compiler_params=pltpu.CompilerParams(needs_layout_passes=False),
    )


def _mmT(w, x):
    return lax.dot_general(
        w.astype(jnp.bfloat16),
        x.astype(jnp.bfloat16),
        (((1,), (0,)), ((), ())),
        preferred_element_type=jnp.float32,
    )


def _dense_body(x_ref, s_ref, dw1, db1, dw2, db2, dw3, db3,
                vd, vs, wd, ws, bd, bs,
                o1d, o1s, ob1, ow2, ob2, ow3t, ob3, out_ref):
    zero = jnp.float32(0.0)
    x = x_ref[...]
    h = jnp.maximum(_mmT(dw1[...], x) + db1[...], zero)
    h = jnp.maximum(_mmT(dw2[...], h) + db2[...], zero)
    d = jnp.maximum(_mmT(dw3[...], h) + db3[...], zero)
    s = s_ref[...]
    xld, xls = d, s
    for l in range(NL):
        xv = _mmT(vd[l], xld) + _mmT(vs[l], xls)
        xld = d * (_mmT(wd[l], xv) + bd[l]) + xld
        xls = s * (_mmT(ws[l], xv) + bs[l]) + xls
    h = jnp.maximum(_mmT(o1d[...], xld) + _mmT(o1s[...], xls) + ob1[...],
                    zero)
    h = jnp.maximum(_mmT(ow2[...], h) + ob2[...], zero)
    out_ref[...] = jnp.sum(h * ow3t[...], axis=0, keepdims=True) + ob3[...]


_BB = 512
_GRID = B // _BB


def _full(shape):
    return pl.BlockSpec(shape, lambda i: (0,) * len(shape))


_dense_call = pl.pallas_call(
    _dense_body,
    grid=(_GRID,),
    in_specs=[
        pl.BlockSpec((DENSE_IN, _BB), lambda i: (0, i)),
        pl.BlockSpec((S_DIM, _BB), lambda i: (0, i)),
        _full((512, DENSE_IN)), _full((512, 1)),
        _full((256, 512)), _full((256, 1)),
        _full((D, 256)), _full((D, 1)),
        _full((NL, LR, D)), _full((NL, LR, S_DIM)),
        _full((NL, D, LR)), _full((NL, S_DIM, LR)),
        _full((NL, D, 1)), _full((NL, S_DIM, 1)),
        _full((512, D)), _full((512, S_DIM)), _full((512, 1)),
        _full((256, 512)), _full((256, 1)),
        _full((256, 1)), _full((1, 1)),
    ],
    out_specs=pl.BlockSpec((1, _BB), lambda i: (0, i)),
    out_shape=jax.ShapeDtypeStruct((1, B), jnp.float32),
)


def kernel(dense_features, sparse_indices, tables, dw1, db1, dw2, db2, dw3,
           db3, cnV, cnW, cnB, ow1, ob1, ow2, ob2, ow3, ob3):
    idx_t = sparse_indices.T
    tables_t = tables.transpose(0, 2, 1)
    s_t = _sc_gather()(idx_t, tables_t)

    vd, vs = cnV[:, :, :D], cnV[:, :, D:]
    wd, ws = cnW[:, :D, :], cnW[:, D:, :]
    bd, bs = cnB[:, :D, None], cnB[:, D:, None]
    o1d, o1s = ow1[:, :D], ow1[:, D:]

    logits_t = _dense_call(
        dense_features.T, s_t,
        dw1, db1[:, None], dw2, db2[:, None], dw3, db3[:, None],
        vd, vs, wd, ws, bd, bs,
        o1d, o1s, ob1[:, None], ow2, ob2[:, None], ow3.T, ob3[:, None],
    )
    return logits_t.reshape(B, 1)


# speedup vs baseline: 5.5482x; 1.0397x over previous
import functools

import jax
import jax.numpy as jnp
from jax import lax
from jax.experimental import pallas as pl
from jax.experimental.pallas import tpu as pltpu
from jax.experimental.pallas import tpu_sc as plsc

F = 26
V = 100000
D = 32
B = 4096
DENSE_IN = 13
LR = 512
NL = 3
CROSS_IN = (F + 1) * D
S_DIM = F * D

_NC = 2
_NS = 16
_NW = _NC * _NS
_GROUPS = B // 16


_H0 = 49920
_H1 = V - _H0


def _sc_gather_body(idx_hbm, table_hbm, out_hbm,
                    bufa, bufb, idxa, idxb, resa, resb,
                    rs0, rs1, is0, is1, os0, os1):
    w = lax.axis_index("s") * _NC + lax.axis_index("c")
    bufs, rsem = (bufa, bufb), (rs0, rs1)
    idxv, isem = (idxa, idxb), (is0, is1)
    resv, osem = (resa, resb), (os0, os1)

    def start_row(k):
        f, h = divmod(k, 2)
        off, ln = (0, _H0) if h == 0 else (_H0, _H1)
        return pltpu.async_copy(
            table_hbm.at[f, w, pl.ds(off, ln)],
            bufs[k % 2].at[pl.ds(0, ln)], rsem[k % 2])

    def start_idx(f):
        return pltpu.async_copy(idx_hbm.at[f], idxv[f % 2], isem[f % 2])

    pend_row = {0: start_row(0)}
    pend_idx = {0: start_idx(0)}
    pend_out = {}
    for k in range(2 * F):
        f, h = divmod(k, 2)
        if k + 1 < 2 * F:
            pend_row[k + 1] = start_row(k + 1)
        if h == 0 and f + 1 < F:
            pend_idx[f + 1] = start_idx(f + 1)
        pend_row.pop(k).wait()
        if h == 0:
            pend_idx.pop(f).wait()
            if f >= 2:
                pend_out.pop(f - 2).wait()
        buf, iv, rv = bufs[k % 2], idxv[f % 2], resv[f % 2]

        if h == 0:
            @plsc.parallel_loop(0, B, step=16, unroll=4)
            def body0(g, iv=iv, rv=rv, buf=buf):
                idx16 = iv[pl.ds(g, 16)]
                m = idx16 < _H0
                rv[pl.ds(g, 16)] = plsc.load_gather(buf, [idx16], mask=m)
        else:
            @plsc.parallel_loop(0, B, step=16, unroll=4)
            def body1(g, iv=iv, rv=rv, buf=buf):
                idx16 = iv[pl.ds(g, 16)]
                m = idx16 >= _H0
                gv = plsc.load_gather(buf, [idx16 - _H0], mask=m)
                rv[pl.ds(g, 16)] = jnp.where(m, gv, rv[pl.ds(g, 16)])
            pend_out[f] = pltpu.async_copy(rv, out_hbm.at[f * D + w],
                                           osem[f % 2])
    pend_out.pop(F - 2).wait()
    pend_out.pop(F - 1).wait()


@functools.cache
def _sc_gather():
    return pl.kernel(
        _sc_gather_body,
        out_type=jax.ShapeDtypeStruct((S_DIM, B), jnp.float32),
        mesh=plsc.VectorSubcoreMesh(core_axis_name="c", subcore_axis_name="s"),
        scratch_types=[
            pltpu.VMEM((_H1,), jnp.float32),
            pltpu.VMEM((_H1,), jnp.float32),
            pltpu.VMEM((B,), jnp.int32),
            pltpu.VMEM((B,), jnp.int32),
            pltpu.VMEM((B,), jnp.float32),
            pltpu.VMEM((B,), jnp.float32),
            pltpu.SemaphoreType.DMA,
            pltpu.SemaphoreType.DMA,
            pltpu.SemaphoreType.DMA,
            pltpu.SemaphoreType.DMA,
            pltpu.SemaphoreType.DMA,
            pltpu.SemaphoreType.DMA,
        ],
        compiler_params=pltpu.CompilerParams(needs_layout_passes=False),
    )


def _mmT(w, x):
    return lax.dot_general(
        w.astype(jnp.bfloat16),
        x.astype(jnp.bfloat16),
        (((1,), (0,)), ((), ())),
        preferred_element_type=jnp.float32,
    )


def _dense_body(x_ref, s_ref, dw1, db1, dw2, db2, dw3, db3,
                vd, vs, wd, ws, bd, bs,
                o1d, o1s, ob1, ow2, ob2, ow3t, ob3, out_ref):
    zero = jnp.float32(0.0)
    x = x_ref[...]
    h = jnp.maximum(_mmT(dw1[...], x) + db1[...], zero)
    h = jnp.maximum(_mmT(dw2[...], h) + db2[...], zero)
    d = jnp.maximum(_mmT(dw3[...], h) + db3[...], zero)
    s = s_ref[...]
    xld, xls = d, s
    for l in range(NL):
        xv = _mmT(vd[l], xld) + _mmT(vs[l], xls)
        xld = d * (_mmT(wd[l], xv) + bd[l]) + xld
        xls = s * (_mmT(ws[l], xv) + bs[l]) + xls
    h = jnp.maximum(_mmT(o1d[...], xld) + _mmT(o1s[...], xls) + ob1[...],
                    zero)
    h = jnp.maximum(_mmT(ow2[...], h) + ob2[...], zero)
    out_ref[...] = jnp.sum(h * ow3t[...], axis=0, keepdims=True) + ob3[...]


_BB = 512
_GRID = B // _BB


def _full(shape):
    return pl.BlockSpec(shape, lambda i: (0,) * len(shape))


_dense_call = pl.pallas_call(
    _dense_body,
    grid=(_GRID,),
    in_specs=[
        pl.BlockSpec((DENSE_IN, _BB), lambda i: (0, i)),
        pl.BlockSpec((S_DIM, _BB), lambda i: (0, i)),
        _full((512, DENSE_IN)), _full((512, 1)),
        _full((256, 512)), _full((256, 1)),
        _full((D, 256)), _full((D, 1)),
        _full((NL, LR, D)), _full((NL, LR, S_DIM)),
        _full((NL, D, LR)), _full((NL, S_DIM, LR)),
        _full((NL, D, 1)), _full((NL, S_DIM, 1)),
        _full((512, D)), _full((512, S_DIM)), _full((512, 1)),
        _full((256, 512)), _full((256, 1)),
        _full((256, 1)), _full((1, 1)),
    ],
    out_specs=pl.BlockSpec((1, _BB), lambda i: (0, i)),
    out_shape=jax.ShapeDtypeStruct((1, B), jnp.float32),
)


def kernel(dense_features, sparse_indices, tables, dw1, db1, dw2, db2, dw3,
           db3, cnV, cnW, cnB, ow1, ob1, ow2, ob2, ow3, ob3):
    idx_t = sparse_indices.T
    tables_t = tables.transpose(0, 2, 1)
    s_t = _sc_gather()(idx_t, tables_t)

    vd, vs = cnV[:, :, :D], cnV[:, :, D:]
    wd, ws = cnW[:, :D, :], cnW[:, D:, :]
    bd, bs = cnB[:, :D, None], cnB[:, D:, None]
    o1d, o1s = ow1[:, :D], ow1[:, D:]

    logits_t = _dense_call(
        dense_features.T, s_t,
        dw1, db1[:, None], dw2, db2[:, None], dw3, db3[:, None],
        vd, vs, wd, ws, bd, bs,
        o1d, o1s, ob1[:, None], ow2, ob2[:, None], ow3.T, ob3[:, None],
    )
    return logits_t.reshape(B, 1)


# speedup vs baseline: 5.5799x; 1.0057x over previous
import functools

import jax
import jax.numpy as jnp
from jax import lax
from jax.experimental import pallas as pl
from jax.experimental.pallas import tpu as pltpu
from jax.experimental.pallas import tpu_sc as plsc

F = 26
V = 100000
D = 32
B = 4096
DENSE_IN = 13
LR = 512
NL = 3
CROSS_IN = (F + 1) * D
S_DIM = F * D

_NC = 2
_NS = 16
_NW = _NC * _NS
_GROUPS = B // 16


_H0 = 49920
_H1 = V - _H0


def _sc_gather_body(idx_hbm, table_hbm, out_hbm,
                    bufa, bufb, idxa, idxb, resa, resb,
                    rs0, rs1, is0, is1, os0, os1):
    w = lax.axis_index("s") * _NC + lax.axis_index("c")
    bufs, rsem = (bufa, bufb), (rs0, rs1)
    idxv, isem = (idxa, idxb), (is0, is1)
    resv, osem = (resa, resb), (os0, os1)

    def start_row(k):
        f, h = divmod(k, 2)
        off, ln = (0, _H0) if h == 0 else (_H0, _H1)
        return pltpu.async_copy(
            table_hbm.at[f, w, pl.ds(off, ln)],
            bufs[k % 2].at[pl.ds(0, ln)], rsem[k % 2])

    def start_idx(f):
        return pltpu.async_copy(idx_hbm.at[f], idxv[f % 2], isem[f % 2])

    pend_row = {0: start_row(0)}
    pend_idx = {0: start_idx(0)}
    pend_out = {}
    for k in range(2 * F):
        f, h = divmod(k, 2)
        if k + 1 < 2 * F:
            pend_row[k + 1] = start_row(k + 1)
        if h == 0 and f + 1 < F:
            pend_idx[f + 1] = start_idx(f + 1)
        pend_row.pop(k).wait()
        if h == 0:
            pend_idx.pop(f).wait()
            if f >= 2:
                pend_out.pop(f - 2).wait()
        buf, iv, rv = bufs[k % 2], idxv[f % 2], resv[f % 2]

        lanes = lax.broadcasted_iota(jnp.int32, (16,), 0)
        if h == 0:
            @plsc.parallel_loop(0, B, step=16, unroll=8)
            def body0(g, iv=iv, rv=rv, buf=buf, lanes=lanes):
                idx16 = iv[pl.ds(g, 16)]
                m = idx16 < _H0
                gv = plsc.load_gather(buf, [idx16], mask=m)
                plsc.store_scatter(rv, [lanes + g], gv, mask=m)
        else:
            @plsc.parallel_loop(0, B, step=16, unroll=8)
            def body1(g, iv=iv, rv=rv, buf=buf, lanes=lanes):
                idx16 = iv[pl.ds(g, 16)]
                m = idx16 >= _H0
                gv = plsc.load_gather(buf, [idx16 - _H0], mask=m)
                plsc.store_scatter(rv, [lanes + g], gv, mask=m)
            pend_out[f] = pltpu.async_copy(rv, out_hbm.at[f * D + w],
                                           osem[f % 2])
    pend_out.pop(F - 2).wait()
    pend_out.pop(F - 1).wait()


@functools.cache
def _sc_gather():
    return pl.kernel(
        _sc_gather_body,
        out_type=jax.ShapeDtypeStruct((S_DIM, B), jnp.float32),
        mesh=plsc.VectorSubcoreMesh(core_axis_name="c", subcore_axis_name="s"),
        scratch_types=[
            pltpu.VMEM((_H1,), jnp.float32),
            pltpu.VMEM((_H1,), jnp.float32),
            pltpu.VMEM((B,), jnp.int32),
            pltpu.VMEM((B,), jnp.int32),
            pltpu.VMEM((B,), jnp.float32),
            pltpu.VMEM((B,), jnp.float32),
            pltpu.SemaphoreType.DMA,
            pltpu.SemaphoreType.DMA,
            pltpu.SemaphoreType.DMA,
            pltpu.SemaphoreType.DMA,
            pltpu.SemaphoreType.DMA,
            pltpu.SemaphoreType.DMA,
        ],
        compiler_params=pltpu.CompilerParams(needs_layout_passes=False),
    )


def _mmT(w, x):
    return lax.dot_general(
        w.astype(jnp.bfloat16),
        x.astype(jnp.bfloat16),
        (((1,), (0,)), ((), ())),
        preferred_element_type=jnp.float32,
    )


def _dense_body(x_ref, s_ref, dw1, db1, dw2, db2, dw3, db3,
                vd, vs, wd, ws, bd, bs,
                o1d, o1s, ob1, ow2, ob2, ow3t, ob3, out_ref):
    zero = jnp.float32(0.0)
    x = x_ref[...]
    h = jnp.maximum(_mmT(dw1[...], x) + db1[...], zero)
    h = jnp.maximum(_mmT(dw2[...], h) + db2[...], zero)
    d = jnp.maximum(_mmT(dw3[...], h) + db3[...], zero)
    s = s_ref[...]
    xld, xls = d, s
    for l in range(NL):
        xv = _mmT(vd[l], xld) + _mmT(vs[l], xls)
        xld = d * (_mmT(wd[l], xv) + bd[l]) + xld
        xls = s * (_mmT(ws[l], xv) + bs[l]) + xls
    h = jnp.maximum(_mmT(o1d[...], xld) + _mmT(o1s[...], xls) + ob1[...],
                    zero)
    h = jnp.maximum(_mmT(ow2[...], h) + ob2[...], zero)
    out_ref[...] = jnp.sum(h * ow3t[...], axis=0, keepdims=True) + ob3[...]


_BB = 1024
_GRID = B // _BB


def _full(shape):
    return pl.BlockSpec(shape, lambda i: (0,) * len(shape))


_dense_call = pl.pallas_call(
    _dense_body,
    grid=(_GRID,),
    in_specs=[
        pl.BlockSpec((DENSE_IN, _BB), lambda i: (0, i)),
        pl.BlockSpec((S_DIM, _BB), lambda i: (0, i)),
        _full((512, DENSE_IN)), _full((512, 1)),
        _full((256, 512)), _full((256, 1)),
        _full((D, 256)), _full((D, 1)),
        _full((NL, LR, D)), _full((NL, LR, S_DIM)),
        _full((NL, D, LR)), _full((NL, S_DIM, LR)),
        _full((NL, D, 1)), _full((NL, S_DIM, 1)),
        _full((512, D)), _full((512, S_DIM)), _full((512, 1)),
        _full((256, 512)), _full((256, 1)),
        _full((256, 1)), _full((1, 1)),
    ],
    out_specs=pl.BlockSpec((1, _BB), lambda i: (0, i)),
    out_shape=jax.ShapeDtypeStruct((1, B), jnp.float32),
)


def kernel(dense_features, sparse_indices, tables, dw1, db1, dw2, db2, dw3,
           db3, cnV, cnW, cnB, ow1, ob1, ow2, ob2, ow3, ob3):
    idx_t = sparse_indices.T
    tables_t = tables.transpose(0, 2, 1)
    s_t = _sc_gather()(idx_t, tables_t)

    vd, vs = cnV[:, :, :D], cnV[:, :, D:]
    wd, ws = cnW[:, :D, :], cnW[:, D:, :]
    bd, bs = cnB[:, :D, None], cnB[:, D:, None]
    o1d, o1s = ow1[:, :D], ow1[:, D:]

    logits_t = _dense_call(
        dense_features.T, s_t,
        dw1, db1[:, None], dw2, db2[:, None], dw3, db3[:, None],
        vd, vs, wd, ws, bd, bs,
        o1d, o1s, ob1[:, None], ow2, ob2[:, None], ow3.T, ob3[:, None],
    )
    return logits_t.reshape(B, 1)


# speedup vs baseline: 5.9672x; 1.0694x over previous
import functools

import jax
import jax.numpy as jnp
from jax import lax
from jax.experimental import pallas as pl
from jax.experimental.pallas import tpu as pltpu
from jax.experimental.pallas import tpu_sc as plsc

F = 26
V = 100000
D = 32
B = 4096
DENSE_IN = 13
LR = 512
NL = 3
CROSS_IN = (F + 1) * D
S_DIM = F * D

_NC = 2
_NS = 16
_NW = _NC * _NS
_GROUPS = B // 16


_NQ = 4
_QS = 24960
_QL = 25120
_DEPTH = 3


def _sc_gather_body(idx_hbm, table_hbm, out_hbm,
                    buf0, buf1, buf2, buf3, idxa, idxb, resa, resb,
                    rs0, rs1, rs2, rs3, is0, is1, os0, os1):
    w = lax.axis_index("s") * _NC + lax.axis_index("c")
    bufs, rsem = (buf0, buf1, buf2, buf3), (rs0, rs1, rs2, rs3)
    idxv, isem = (idxa, idxb), (is0, is1)
    resv, osem = (resa, resb), (os0, os1)
    n_items = _NQ * F

    def start_row(k):
        f, q = divmod(k, _NQ)
        ln = _QS if q < _NQ - 1 else _QL
        return pltpu.async_copy(
            table_hbm.at[f, w, pl.ds(q * _QS, ln)],
            bufs[k % _NQ].at[pl.ds(0, ln)], rsem[k % _NQ])

    def start_idx(f):
        return pltpu.async_copy(idx_hbm.at[f], idxv[f % 2], isem[f % 2])

    pend_row = {k: start_row(k) for k in range(_DEPTH)}
    pend_idx = {0: start_idx(0)}
    pend_out = {}
    lanes = lax.broadcasted_iota(jnp.int32, (16,), 0)
    for k in range(n_items):
        f, q = divmod(k, _NQ)
        if k + _DEPTH < n_items:
            pend_row[k + _DEPTH] = start_row(k + _DEPTH)
        if q == 0 and f + 1 < F:
            pend_idx[f + 1] = start_idx(f + 1)
        pend_row.pop(k).wait()
        if q == 0:
            pend_idx.pop(f).wait()
            if f >= 2:
                pend_out.pop(f - 2).wait()
        buf, iv, rv = bufs[k % _NQ], idxv[f % 2], resv[f % 2]
        lo = q * _QS

        @plsc.parallel_loop(0, B, step=16, unroll=4)
        def body(g, iv=iv, rv=rv, buf=buf, lanes=lanes, lo=lo, q=q):
            idx16 = iv[pl.ds(g, 16)]
            if q == 0:
                m = idx16 < _QS
            elif q == _NQ - 1:
                m = idx16 >= lo
            else:
                m = (idx16 >= lo) & (idx16 < lo + _QS)
            gv = plsc.load_gather(buf, [idx16 - lo], mask=m)
            plsc.store_scatter(rv, [lanes + g], gv, mask=m)

        if q == _NQ - 1:
            pend_out[f] = pltpu.async_copy(rv, out_hbm.at[f * D + w],
                                           osem[f % 2])
    pend_out.pop(F - 2).wait()
    pend_out.pop(F - 1).wait()


@functools.cache
def _sc_gather():
    return pl.kernel(
        _sc_gather_body,
        out_type=jax.ShapeDtypeStruct((S_DIM, B), jnp.float32),
        mesh=plsc.VectorSubcoreMesh(core_axis_name="c", subcore_axis_name="s"),
        scratch_types=(
            [pltpu.VMEM((_QL,), jnp.float32)] * 4
            + [pltpu.VMEM((B,), jnp.int32)] * 2
            + [pltpu.VMEM((B,), jnp.float32)] * 2
            + [pltpu.SemaphoreType.DMA] * 8
        ),
        compiler_params=pltpu.CompilerParams(needs_layout_passes=False),
    )


def _mmT(w, x):
    return lax.dot_general(
        w.astype(jnp.bfloat16),
        x.astype(jnp.bfloat16),
        (((1,), (0,)), ((), ())),
        preferred_element_type=jnp.float32,
    )


def _dense_body(x_ref, s_ref, dw1, db1, dw2, db2, dw3, db3,
                vd, vs, wd, ws, bd, bs,
                o1d, o1s, ob1, ow2, ob2, ow3t, ob3, out_ref):
    zero = jnp.float32(0.0)
    x = x_ref[...]
    h = jnp.maximum(_mmT(dw1[...], x) + db1[...], zero)
    h = jnp.maximum(_mmT(dw2[...], h) + db2[...], zero)
    d = jnp.maximum(_mmT(dw3[...], h) + db3[...], zero)
    s = s_ref[...]
    xld, xls = d, s
    for l in range(NL):
        xv = _mmT(vd[l], xld) + _mmT(vs[l], xls)
        xld = d * (_mmT(wd[l], xv) + bd[l]) + xld
        xls = s * (_mmT(ws[l], xv) + bs[l]) + xls
    h = jnp.maximum(_mmT(o1d[...], xld) + _mmT(o1s[...], xls) + ob1[...],
                    zero)
    h = jnp.maximum(_mmT(ow2[...], h) + ob2[...], zero)
    out_ref[...] = jnp.sum(h * ow3t[...], axis=0, keepdims=True) + ob3[...]


_BB = 1024
_GRID = B // _BB


def _full(shape):
    return pl.BlockSpec(shape, lambda i: (0,) * len(shape))


_dense_call = pl.pallas_call(
    _dense_body,
    grid=(_GRID,),
    in_specs=[
        pl.BlockSpec((DENSE_IN, _BB), lambda i: (0, i)),
        pl.BlockSpec((S_DIM, _BB), lambda i: (0, i)),
        _full((512, DENSE_IN)), _full((512, 1)),
        _full((256, 512)), _full((256, 1)),
        _full((D, 256)), _full((D, 1)),
        _full((NL, LR, D)), _full((NL, LR, S_DIM)),
        _full((NL, D, LR)), _full((NL, S_DIM, LR)),
        _full((NL, D, 1)), _full((NL, S_DIM, 1)),
        _full((512, D)), _full((512, S_DIM)), _full((512, 1)),
        _full((256, 512)), _full((256, 1)),
        _full((256, 1)), _full((1, 1)),
    ],
    out_specs=pl.BlockSpec((1, _BB), lambda i: (0, i)),
    out_shape=jax.ShapeDtypeStruct((1, B), jnp.float32),
)


def kernel(dense_features, sparse_indices, tables, dw1, db1, dw2, db2, dw3,
           db3, cnV, cnW, cnB, ow1, ob1, ow2, ob2, ow3, ob3):
    idx_t = sparse_indices.T
    tables_t = tables.transpose(0, 2, 1)
    s_t = _sc_gather()(idx_t, tables_t)

    vd, vs = cnV[:, :, :D], cnV[:, :, D:]
    wd, ws = cnW[:, :D, :], cnW[:, D:, :]
    bd, bs = cnB[:, :D, None], cnB[:, D:, None]
    o1d, o1s = ow1[:, :D], ow1[:, D:]

    logits_t = _dense_call(
        dense_features.T, s_t,
        dw1, db1[:, None], dw2, db2[:, None], dw3, db3[:, None],
        vd, vs, wd, ws, bd, bs,
        o1d, o1s, ob1[:, None], ow2, ob2[:, None], ow3.T, ob3[:, None],
    )
    return logits_t.reshape(B, 1)
